# baseline (speedup 1.0000x reference)
import functools

import jax
import jax.numpy as jnp
from jax import lax
from jax.experimental import pallas as pl
from jax.experimental.pallas import tpu as pltpu

EPSILON = 1e-6
MATCH_THRESHOLD = 0.5
RADIUS_THRESHOLD = 0.05

NR = 1024
SC = 2048
NC = 4


def _prep_kernel(wobs_ref, w1_ref, w3_ref, bel_ref, mask_ref,
                 wall_ref, angsT_ref, andm_ref, orm_ref):
    D = wobs_ref.shape[0]
    Hq = w1_ref.shape[0]
    S = bel_ref.shape[0]
    wall_ref[:, :D] = wobs_ref[...].astype(jnp.bfloat16).T
    wall_ref[:, D:D + Hq] = w1_ref[...].astype(jnp.bfloat16).T
    wall_ref[:, D + Hq:] = w3_ref[...].astype(jnp.bfloat16).T
    belT = bel_ref[...].T
    n2 = jnp.sum(belT * belT, axis=0, keepdims=True)
    r = 1.0 / jnp.maximum(jnp.sqrt(n2), EPSILON)
    angsT = (belT * r).astype(jnp.bfloat16)
    active = mask_ref[...] != 0
    revcol = (S - 1) - lax.broadcasted_iota(jnp.int32, (1, S), 1)
    andm = jnp.where(active, jnp.int32(-8192), jnp.int32(0))
    orm = revcol | jnp.where(active, jnp.int32(0), jnp.int32(-2147483648))
    for c in range(NC):
        cols = slice(c * SC, (c + 1) * SC)
        angsT_ref[c, :, :] = angsT[:, cols]
        andm_ref[c, 0, :] = andm[0, cols]
        orm_ref[c, 0, :] = orm[0, cols]


def _packed(raw, andm, orm):
    b = lax.bitcast_convert_type(raw, jnp.int32)
    return lax.bitcast_convert_type((b & andm) | orm, jnp.float32)


def _main_kernel(hid_ref, wall_ref, wbd_ref, b13_ref, b24_ref, angsT_ref,
                 andm_ref, orm_ref, obsb_ref, slots_ref, simsout_ref,
                 obs_ref, gp_ref, rinv_ref, rawA_ref, rawB_ref, mbest_ref):
    c = pl.program_id(1)
    S = SC * NC

    @pl.when(c == 0)
    def _featurize():
        hb = hid_ref[...].astype(jnp.bfloat16)
        acc = jnp.dot(hb, wall_ref[...], preferred_element_type=jnp.float32)
        obs = acc[:, :256]
        h13 = jnp.maximum(acc[:, 256:] + b13_ref[...], 0.0)
        gl = lax.dot_general(h13.astype(jnp.bfloat16), wbd_ref[...],
                             (((1,), (1,)), ((), ())),
                             preferred_element_type=jnp.float32) + b24_ref[...]
        gate = jax.nn.sigmoid(gl[:, 0:1])
        prec = jax.nn.softplus(gl[:, 1:2])
        gp = gate * prec
        onorm = jnp.sqrt(jnp.sum(obs * obs, axis=1, keepdims=True))
        rinv = 1.0 / jnp.maximum(onorm, EPSILON)
        obsb_ref[...] = obs * (rinv * gp)
        obs_ref[...] = obs.astype(jnp.bfloat16)
        gp_ref[...] = gp
        rinv_ref[...] = rinv

    @pl.when(c % 2 == 0)
    def _dot_even():
        rawA_ref[...] = jnp.dot(obs_ref[...], angsT_ref[0],
                                preferred_element_type=jnp.float32)

    @pl.when(c % 2 == 1)
    def _dot_odd():
        rawB_ref[...] = jnp.dot(obs_ref[...], angsT_ref[0],
                                preferred_element_type=jnp.float32)

    @pl.when(c == 1)
    def _tail_init():
        pf = _packed(rawA_ref[...], andm_ref[0], orm_ref[0])
        mbest_ref[...] = jnp.max(pf, axis=1, keepdims=True)

    @pl.when(c == 3)
    def _tail_even():
        pf = _packed(rawA_ref[...], andm_ref[0], orm_ref[0])
        mbest_ref[...] = jnp.maximum(mbest_ref[...],
                                     jnp.max(pf, axis=1, keepdims=True))

    @pl.when((c == 2) | (c == 4))
    def _tail_odd():
        pf = _packed(rawB_ref[...], andm_ref[0], orm_ref[0])
        mbest_ref[...] = jnp.maximum(mbest_ref[...],
                                     jnp.max(pf, axis=1, keepdims=True))

    @pl.when(c == NC)
    def _finalize():
        pbest = lax.bitcast_convert_type(mbest_ref[...][:, 0], jnp.int32)
        bidx = (S - 1) - (pbest & jnp.int32(8191))
        bestv = lax.bitcast_convert_type(pbest & jnp.int32(-8192),
                                         jnp.float32) * rinv_ref[...][:, 0]
        matched = (gp_ref[...][:, 0] > RADIUS_THRESHOLD) & \
                  (bestv > MATCH_THRESHOLD)
        slots_ref[0, 0, :] = jnp.where(matched, bidx, -1).astype(jnp.int32)
        simsout_ref[0, 0, :] = jnp.where(matched, bestv, 0.0)


@functools.partial(jax.jit, static_argnames=())
def kernel(hidden, beliefs, active_mask, W_obs, w1, b1, w2, b2, w3, b3, w4, b4):
    B, T, H = hidden.shape
    D = W_obs.shape[0]
    Hq = w1.shape[0]
    S = beliefs.shape[0]
    N = B * T
    nrow = N // NR

    hid2d = hidden.reshape(N, H)
    wbd = jnp.zeros((2, 2 * Hq), jnp.float32)
    wbd = wbd.at[0, :Hq].set(w2[0]).at[1, Hq:].set(w4[0]).astype(jnp.bfloat16)
    b13 = jnp.concatenate([b1, b3]).reshape(1, 2 * Hq).astype(jnp.float32)
    b24 = jnp.concatenate([b2, b4]).reshape(1, 2).astype(jnp.float32)
    maski = active_mask.astype(jnp.int32).reshape(1, S)

    wall, angsT4, andm4, orm4 = pl.pallas_call(
        _prep_kernel,
        out_shape=[
            jax.ShapeDtypeStruct((H, D + 2 * Hq), jnp.bfloat16),
            jax.ShapeDtypeStruct((NC, D, SC), jnp.bfloat16),
            jax.ShapeDtypeStruct((NC, 1, SC), jnp.int32),
            jax.ShapeDtypeStruct((NC, 1, SC), jnp.int32),
        ],
    )(W_obs, w1, w3, beliefs, maski)

    last = NC - 1
    obsb, slots3, sims3 = pl.pallas_call(
        _main_kernel,
        grid=(nrow, NC + 1),
        in_specs=[
            pl.BlockSpec((NR, H), lambda r, c: (r, 0)),
            pl.BlockSpec((H, D + 2 * Hq), lambda r, c: (0, 0)),
            pl.BlockSpec((2, 2 * Hq), lambda r, c: (0, 0)),
            pl.BlockSpec((1, 2 * Hq), lambda r, c: (0, 0)),
            pl.BlockSpec((1, 2), lambda r, c: (0, 0)),
            pl.BlockSpec((1, D, SC), lambda r, c: (jnp.minimum(c, last), 0, 0)),
            pl.BlockSpec((1, 1, SC),
                         lambda r, c: (jnp.clip(c - 1, 0, last), 0, 0)),
            pl.BlockSpec((1, 1, SC),
                         lambda r, c: (jnp.clip(c - 1, 0, last), 0, 0)),
        ],
        out_specs=[
            pl.BlockSpec((NR, D), lambda r, c: (r, 0)),
            pl.BlockSpec((1, 1, NR), lambda r, c: (r, 0, 0)),
            pl.BlockSpec((1, 1, NR), lambda r, c: (r, 0, 0)),
        ],
        out_shape=[
            jax.ShapeDtypeStruct((N, D), jnp.float32),
            jax.ShapeDtypeStruct((nrow, 1, NR), jnp.int32),
            jax.ShapeDtypeStruct((nrow, 1, NR), jnp.float32),
        ],
        scratch_shapes=[
            pltpu.VMEM((NR, D), jnp.bfloat16),
            pltpu.VMEM((NR, 1), jnp.float32),
            pltpu.VMEM((NR, 1), jnp.float32),
            pltpu.VMEM((NR, SC), jnp.float32),
            pltpu.VMEM((NR, SC), jnp.float32),
            pltpu.VMEM((NR, 1), jnp.float32),
        ],
    )(hid2d, wall, wbd, b13, b24, angsT4, andm4, orm4)

    return (obsb.reshape(B, T, D), slots3.reshape(N), sims3.reshape(N))


# speedup vs baseline: 1.4130x; 1.4130x over previous
import functools

import jax
import jax.numpy as jnp
from jax import lax
from jax.experimental import pallas as pl
from jax.experimental.pallas import tpu as pltpu

EPSILON = 1e-6
MATCH_THRESHOLD = 0.5
RADIUS_THRESHOLD = 0.05

NB = 1024
SUB = 2


def _prep_kernel(wobs_ref, w1_ref, w3_ref, bel_ref, mask_ref,
                 wall_ref, angsT_ref, andm_ref, orm_ref):
    D = wobs_ref.shape[0]
    Hq = w1_ref.shape[0]
    S = bel_ref.shape[0]
    wall_ref[:, :D] = wobs_ref[...].astype(jnp.bfloat16).T
    wall_ref[:, D:D + Hq] = w1_ref[...].astype(jnp.bfloat16).T
    wall_ref[:, D + Hq:] = w3_ref[...].astype(jnp.bfloat16).T
    belT = bel_ref[...].T
    n2 = jnp.sum(belT * belT, axis=0, keepdims=True)
    r = 1.0 / jnp.maximum(jnp.sqrt(n2), EPSILON)
    angsT_ref[...] = (belT * r).astype(jnp.bfloat16)
    active = mask_ref[...] != 0
    revcol = (S - 1) - lax.broadcasted_iota(jnp.int32, (1, S), 1)
    andm_ref[...] = jnp.where(active, jnp.int32(-8192), jnp.int32(0))
    orm_ref[...] = revcol | jnp.where(active, jnp.int32(0),
                                      jnp.int32(-2147483648))


def _main_kernel(hid_ref, wall_ref, wbd_ref, b13_ref, b24_ref, angsT_ref,
                 andm_ref, orm_ref, obsb_ref, slots_ref, simsout_ref):
    S = angsT_ref.shape[1]
    for sub in range(SUB):
        rows = slice(sub * NB, (sub + 1) * NB)
        hb = hid_ref[rows, :].astype(jnp.bfloat16)
        acc = jnp.dot(hb, wall_ref[...], preferred_element_type=jnp.float32)
        obs = acc[:, :256]
        h13 = jnp.maximum(acc[:, 256:] + b13_ref[...], 0.0)
        gl = lax.dot_general(h13.astype(jnp.bfloat16), wbd_ref[...],
                             (((1,), (1,)), ((), ())),
                             preferred_element_type=jnp.float32) + b24_ref[...]
        gate = jax.nn.sigmoid(gl[:, 0:1])
        prec = jax.nn.softplus(gl[:, 1:2])
        gp = gate * prec
        onorm = jnp.sqrt(jnp.sum(obs * obs, axis=1, keepdims=True))
        rinv = 1.0 / jnp.maximum(onorm, EPSILON)
        obsb_ref[rows, :] = obs * (rinv * gp)
        raw = jnp.dot(obs.astype(jnp.bfloat16), angsT_ref[...],
                      preferred_element_type=jnp.float32)
        b = lax.bitcast_convert_type(raw, jnp.int32)
        packed = (b & andm_ref[...]) | orm_ref[...]
        pmax = jnp.max(lax.bitcast_convert_type(packed, jnp.float32), axis=1)
        pbest = lax.bitcast_convert_type(pmax, jnp.int32)
        bidx = (S - 1) - (pbest & jnp.int32(8191))
        bestv = lax.bitcast_convert_type(pbest & jnp.int32(-8192),
                                         jnp.float32) * rinv[:, 0]
        matched = (gp[:, 0] > RADIUS_THRESHOLD) & (bestv > MATCH_THRESHOLD)
        slots_ref[sub, 0, :] = jnp.where(matched, bidx, -1).astype(jnp.int32)
        simsout_ref[sub, 0, :] = jnp.where(matched, bestv, 0.0)


@functools.partial(jax.jit, static_argnames=())
def kernel(hidden, beliefs, active_mask, W_obs, w1, b1, w2, b2, w3, b3, w4, b4):
    B, T, H = hidden.shape
    D = W_obs.shape[0]
    Hq = w1.shape[0]
    S = beliefs.shape[0]
    N = B * T
    nstep = N // (NB * SUB)

    hid2d = hidden.reshape(N, H)
    wbd = jnp.zeros((2, 2 * Hq), jnp.float32)
    wbd = wbd.at[0, :Hq].set(w2[0]).at[1, Hq:].set(w4[0]).astype(jnp.bfloat16)
    b13 = jnp.concatenate([b1, b3]).reshape(1, 2 * Hq).astype(jnp.float32)
    b24 = jnp.concatenate([b2, b4]).reshape(1, 2).astype(jnp.float32)
    maski = active_mask.astype(jnp.int32).reshape(1, S)

    wall, angsT, andm, orm = pl.pallas_call(
        _prep_kernel,
        out_shape=[
            jax.ShapeDtypeStruct((H, D + 2 * Hq), jnp.bfloat16),
            jax.ShapeDtypeStruct((D, S), jnp.bfloat16),
            jax.ShapeDtypeStruct((1, S), jnp.int32),
            jax.ShapeDtypeStruct((1, S), jnp.int32),
        ],
    )(W_obs, w1, w3, beliefs, maski)

    obsb, slots3, sims3 = pl.pallas_call(
        _main_kernel,
        grid=(nstep,),
        in_specs=[
            pl.BlockSpec((NB * SUB, H), lambda i: (i, 0)),
            pl.BlockSpec((H, D + 2 * Hq), lambda i: (0, 0)),
            pl.BlockSpec((2, 2 * Hq), lambda i: (0, 0)),
            pl.BlockSpec((1, 2 * Hq), lambda i: (0, 0)),
            pl.BlockSpec((1, 2), lambda i: (0, 0)),
            pl.BlockSpec((D, S), lambda i: (0, 0)),
            pl.BlockSpec((1, S), lambda i: (0, 0)),
            pl.BlockSpec((1, S), lambda i: (0, 0)),
        ],
        out_specs=[
            pl.BlockSpec((NB * SUB, D), lambda i: (i, 0)),
            pl.BlockSpec((SUB, 1, NB), lambda i: (i, 0, 0)),
            pl.BlockSpec((SUB, 1, NB), lambda i: (i, 0, 0)),
        ],
        out_shape=[
            jax.ShapeDtypeStruct((N, D), jnp.float32),
            jax.ShapeDtypeStruct((N // NB, 1, NB), jnp.int32),
            jax.ShapeDtypeStruct((N // NB, 1, NB), jnp.float32),
        ],
    )(hid2d, wall, wbd, b13, b24, angsT, andm, orm)

    return (obsb.reshape(B, T, D), slots3.reshape(N), sims3.reshape(N))


# speedup vs baseline: 1.4377x; 1.0175x over previous
import functools

import jax
import jax.numpy as jnp
from jax import lax
from jax.experimental import pallas as pl
from jax.experimental.pallas import tpu as pltpu

EPSILON = 1e-6
MATCH_THRESHOLD = 0.5
RADIUS_THRESHOLD = 0.05

NB = 256
SUB = 8


def _prep_kernel(wobs_ref, w1_ref, w3_ref, bel_ref, mask_ref,
                 wall_ref, angsT_ref, andm_ref, orm_ref):
    D = wobs_ref.shape[0]
    Hq = w1_ref.shape[0]
    S = bel_ref.shape[0]
    wall_ref[:, :D] = wobs_ref[...].astype(jnp.bfloat16).T
    wall_ref[:, D:D + Hq] = w1_ref[...].astype(jnp.bfloat16).T
    wall_ref[:, D + Hq:] = w3_ref[...].astype(jnp.bfloat16).T
    belT = bel_ref[...].T
    n2 = jnp.sum(belT * belT, axis=0, keepdims=True)
    r = 1.0 / jnp.maximum(jnp.sqrt(n2), EPSILON)
    angsT_ref[...] = (belT * r).astype(jnp.bfloat16)
    active = mask_ref[...] != 0
    revcol = (S - 1) - lax.broadcasted_iota(jnp.int32, (1, S), 1)
    andm_ref[...] = jnp.where(active, jnp.int32(-8192), jnp.int32(0))
    orm_ref[...] = revcol | jnp.where(active, jnp.int32(0),
                                      jnp.int32(-2147483648))


def _main_kernel(hid_ref, wall_ref, wbd_ref, b13_ref, b24_ref, angsT_ref,
                 andm_ref, orm_ref, obsb_ref, slots_ref, simsout_ref):
    S = angsT_ref.shape[1]
    for sub in range(SUB):
        rows = slice(sub * NB, (sub + 1) * NB)
        hb = hid_ref[rows, :].astype(jnp.bfloat16)
        acc = jnp.dot(hb, wall_ref[...], preferred_element_type=jnp.float32)
        obs = acc[:, :256]
        h13 = jnp.maximum(acc[:, 256:] + b13_ref[...], 0.0)
        gl = lax.dot_general(h13.astype(jnp.bfloat16), wbd_ref[...],
                             (((1,), (1,)), ((), ())),
                             preferred_element_type=jnp.float32) + b24_ref[...]
        gate = jax.nn.sigmoid(gl[:, 0:1])
        prec = jax.nn.softplus(gl[:, 1:2])
        gp = gate * prec
        onorm = jnp.sqrt(jnp.sum(obs * obs, axis=1, keepdims=True))
        rinv = 1.0 / jnp.maximum(onorm, EPSILON)
        obsb_ref[rows, :] = obs * (rinv * gp)
        raw = jnp.dot(obs.astype(jnp.bfloat16), angsT_ref[...],
                      preferred_element_type=jnp.float32)
        b = lax.bitcast_convert_type(raw, jnp.int32)
        packed = (b & andm_ref[...]) | orm_ref[...]
        pmax = jnp.max(lax.bitcast_convert_type(packed, jnp.float32), axis=1)
        pbest = lax.bitcast_convert_type(pmax, jnp.int32)
        bidx = (S - 1) - (pbest & jnp.int32(8191))
        bestv = lax.bitcast_convert_type(pbest & jnp.int32(-8192),
                                         jnp.float32) * rinv[:, 0]
        matched = (gp[:, 0] > RADIUS_THRESHOLD) & (bestv > MATCH_THRESHOLD)
        slots_ref[sub, 0, :] = jnp.where(matched, bidx, -1).astype(jnp.int32)
        simsout_ref[sub, 0, :] = jnp.where(matched, bestv, 0.0)


@functools.partial(jax.jit, static_argnames=())
def kernel(hidden, beliefs, active_mask, W_obs, w1, b1, w2, b2, w3, b3, w4, b4):
    B, T, H = hidden.shape
    D = W_obs.shape[0]
    Hq = w1.shape[0]
    S = beliefs.shape[0]
    N = B * T
    nstep = N // (NB * SUB)

    hid2d = hidden.reshape(N, H)
    wbd = jnp.zeros((2, 2 * Hq), jnp.float32)
    wbd = wbd.at[0, :Hq].set(w2[0]).at[1, Hq:].set(w4[0]).astype(jnp.bfloat16)
    b13 = jnp.concatenate([b1, b3]).reshape(1, 2 * Hq).astype(jnp.float32)
    b24 = jnp.concatenate([b2, b4]).reshape(1, 2).astype(jnp.float32)
    maski = active_mask.astype(jnp.int32).reshape(1, S)

    wall, angsT, andm, orm = pl.pallas_call(
        _prep_kernel,
        out_shape=[
            jax.ShapeDtypeStruct((H, D + 2 * Hq), jnp.bfloat16),
            jax.ShapeDtypeStruct((D, S), jnp.bfloat16),
            jax.ShapeDtypeStruct((1, S), jnp.int32),
            jax.ShapeDtypeStruct((1, S), jnp.int32),
        ],
    )(W_obs, w1, w3, beliefs, maski)

    obsb, slots3, sims3 = pl.pallas_call(
        _main_kernel,
        grid=(nstep,),
        in_specs=[
            pl.BlockSpec((NB * SUB, H), lambda i: (i, 0)),
            pl.BlockSpec((H, D + 2 * Hq), lambda i: (0, 0)),
            pl.BlockSpec((2, 2 * Hq), lambda i: (0, 0)),
            pl.BlockSpec((1, 2 * Hq), lambda i: (0, 0)),
            pl.BlockSpec((1, 2), lambda i: (0, 0)),
            pl.BlockSpec((D, S), lambda i: (0, 0)),
            pl.BlockSpec((1, S), lambda i: (0, 0)),
            pl.BlockSpec((1, S), lambda i: (0, 0)),
        ],
        out_specs=[
            pl.BlockSpec((NB * SUB, D), lambda i: (i, 0)),
            pl.BlockSpec((SUB, 1, NB), lambda i: (i, 0, 0)),
            pl.BlockSpec((SUB, 1, NB), lambda i: (i, 0, 0)),
        ],
        out_shape=[
            jax.ShapeDtypeStruct((N, D), jnp.float32),
            jax.ShapeDtypeStruct((N // NB, 1, NB), jnp.int32),
            jax.ShapeDtypeStruct((N // NB, 1, NB), jnp.float32),
        ],
    )(hid2d, wall, wbd, b13, b24, angsT, andm, orm)

    return (obsb.reshape(B, T, D), slots3.reshape(N), sims3.reshape(N))


# speedup vs baseline: 1.4471x; 1.0065x over previous
import functools

import jax
import jax.numpy as jnp
from jax import lax
from jax.experimental import pallas as pl
from jax.experimental.pallas import tpu as pltpu

EPSILON = 1e-6
MATCH_THRESHOLD = 0.5
RADIUS_THRESHOLD = 0.05

NB = 512
SUB = 4


def _prep_kernel(wobs_ref, w1_ref, w3_ref, bel_ref, mask_ref,
                 wall_ref, angsT_ref, andm_ref, orm_ref):
    D = wobs_ref.shape[0]
    Hq = w1_ref.shape[0]
    S = bel_ref.shape[0]
    wall_ref[:, :D] = wobs_ref[...].astype(jnp.bfloat16).T
    wall_ref[:, D:D + Hq] = w1_ref[...].astype(jnp.bfloat16).T
    wall_ref[:, D + Hq:] = w3_ref[...].astype(jnp.bfloat16).T
    belT = bel_ref[...].T
    n2 = jnp.sum(belT * belT, axis=0, keepdims=True)
    r = 1.0 / jnp.maximum(jnp.sqrt(n2), EPSILON)
    angsT_ref[...] = (belT * r).astype(jnp.bfloat16)
    active = mask_ref[...] != 0
    revcol = (S - 1) - lax.broadcasted_iota(jnp.int32, (1, S), 1)
    andm_ref[...] = jnp.where(active, jnp.int32(-8192), jnp.int32(0))
    orm_ref[...] = revcol | jnp.where(active, jnp.int32(0),
                                      jnp.int32(-2147483648))


def _main_kernel(hid_ref, wall_ref, wbd_ref, b13_ref, b24_ref, angsT_ref,
                 andm_ref, orm_ref, obsb_ref, slots_ref, simsout_ref):
    S = angsT_ref.shape[1]
    for sub in range(SUB):
        rows = slice(sub * NB, (sub + 1) * NB)
        hb = hid_ref[rows, :].astype(jnp.bfloat16)
        acc = jnp.dot(hb, wall_ref[...], preferred_element_type=jnp.float32)
        obs = acc[:, :256]
        h13 = jnp.maximum(acc[:, 256:] + b13_ref[...], 0.0)
        gl = lax.dot_general(h13.astype(jnp.bfloat16), wbd_ref[...],
                             (((1,), (1,)), ((), ())),
                             preferred_element_type=jnp.float32) + b24_ref[...]
        gate = jax.nn.sigmoid(gl[:, 0:1])
        prec = jax.nn.softplus(gl[:, 1:2])
        gp = gate * prec
        onorm = jnp.sqrt(jnp.sum(obs * obs, axis=1, keepdims=True))
        rinv = 1.0 / jnp.maximum(onorm, EPSILON)
        obsb_ref[rows, :] = obs * (rinv * gp)
        raw = jnp.dot(obs.astype(jnp.bfloat16), angsT_ref[...],
                      preferred_element_type=jnp.float32)
        b = lax.bitcast_convert_type(raw, jnp.int32)
        packed = (b & andm_ref[...]) | orm_ref[...]
        pmax = jnp.max(lax.bitcast_convert_type(packed, jnp.float32), axis=1)
        pbest = lax.bitcast_convert_type(pmax, jnp.int32)
        bidx = (S - 1) - (pbest & jnp.int32(8191))
        bestv = lax.bitcast_convert_type(pbest & jnp.int32(-8192),
                                         jnp.float32) * rinv[:, 0]
        matched = (gp[:, 0] > RADIUS_THRESHOLD) & (bestv > MATCH_THRESHOLD)
        slots_ref[sub, 0, :] = jnp.where(matched, bidx, -1).astype(jnp.int32)
        simsout_ref[sub, 0, :] = jnp.where(matched, bestv, 0.0)


@functools.partial(jax.jit, static_argnames=())
def kernel(hidden, beliefs, active_mask, W_obs, w1, b1, w2, b2, w3, b3, w4, b4):
    B, T, H = hidden.shape
    D = W_obs.shape[0]
    Hq = w1.shape[0]
    S = beliefs.shape[0]
    N = B * T
    nstep = N // (NB * SUB)

    hid2d = hidden.reshape(N, H)
    wbd = jnp.zeros((2, 2 * Hq), jnp.float32)
    wbd = wbd.at[0, :Hq].set(w2[0]).at[1, Hq:].set(w4[0]).astype(jnp.bfloat16)
    b13 = jnp.concatenate([b1, b3]).reshape(1, 2 * Hq).astype(jnp.float32)
    b24 = jnp.concatenate([b2, b4]).reshape(1, 2).astype(jnp.float32)
    maski = active_mask.astype(jnp.int32).reshape(1, S)

    wall, angsT, andm, orm = pl.pallas_call(
        _prep_kernel,
        out_shape=[
            jax.ShapeDtypeStruct((H, D + 2 * Hq), jnp.bfloat16),
            jax.ShapeDtypeStruct((D, S), jnp.bfloat16),
            jax.ShapeDtypeStruct((1, S), jnp.int32),
            jax.ShapeDtypeStruct((1, S), jnp.int32),
        ],
    )(W_obs, w1, w3, beliefs, maski)

    obsb, slots3, sims3 = pl.pallas_call(
        _main_kernel,
        grid=(nstep,),
        in_specs=[
            pl.BlockSpec((NB * SUB, H), lambda i: (i, 0)),
            pl.BlockSpec((H, D + 2 * Hq), lambda i: (0, 0)),
            pl.BlockSpec((2, 2 * Hq), lambda i: (0, 0)),
            pl.BlockSpec((1, 2 * Hq), lambda i: (0, 0)),
            pl.BlockSpec((1, 2), lambda i: (0, 0)),
            pl.BlockSpec((D, S), lambda i: (0, 0)),
            pl.BlockSpec((1, S), lambda i: (0, 0)),
            pl.BlockSpec((1, S), lambda i: (0, 0)),
        ],
        out_specs=[
            pl.BlockSpec((NB * SUB, D), lambda i: (i, 0)),
            pl.BlockSpec((SUB, 1, NB), lambda i: (i, 0, 0)),
            pl.BlockSpec((SUB, 1, NB), lambda i: (i, 0, 0)),
        ],
        out_shape=[
            jax.ShapeDtypeStruct((N, D), jnp.float32),
            jax.ShapeDtypeStruct((N // NB, 1, NB), jnp.int32),
            jax.ShapeDtypeStruct((N // NB, 1, NB), jnp.float32),
        ],
    )(hid2d, wall, wbd, b13, b24, angsT, andm, orm)

    return (obsb.reshape(B, T, D), slots3.reshape(N), sims3.reshape(N))
